# Initial kernel scaffold; baseline (speedup 1.0000x reference)
import jax
import jax.numpy as jnp
from jax.experimental import pallas as pl


def kernel(hidden_states, W_gate, Wg_s, Wu_s, Wd_s, Wg, Wu, Wd, expert_bias):
    raise NotImplementedError("write your pallas kernel here")


# speedup vs baseline: 1.3886x; 1.3886x over previous
import functools

import jax
import jax.numpy as jnp
from jax.experimental import pallas as pl
from jax.experimental.pallas import tpu as pltpu

T = 2048
H = 1024
E = 8
K = 2
INTER = 512
SI = 1024
TBLK = 512


def _router_body(x_ref, wg_ref, b_ref, comb_ref):
    x = x_ref[...]
    scores = jax.nn.sigmoid(
        jnp.dot(x, wg_ref[...], preferred_element_type=jnp.float32))
    biased = scores + b_ref[...]
    iota = jax.lax.broadcasted_iota(jnp.int32, (T, E), 1)
    m0 = jnp.max(biased, axis=1, keepdims=True)
    sel0 = jnp.min(jnp.where(biased >= m0, iota, E), axis=1, keepdims=True)
    neg = jnp.where(iota == sel0, -jnp.inf, biased)
    m1 = jnp.max(neg, axis=1, keepdims=True)
    sel1 = jnp.min(jnp.where(neg >= m1, iota, E), axis=1, keepdims=True)
    s0 = jnp.sum(jnp.where(iota == sel0, scores, 0.0), axis=1, keepdims=True)
    s1 = jnp.sum(jnp.where(iota == sel1, scores, 0.0), axis=1, keepdims=True)
    denom = s0 + s1 + 1e-20
    comb = (jnp.where(iota == sel0, s0, 0.0)
            + jnp.where(iota == sel1, s1, 0.0)) / denom
    comb_ref[...] = comb


def _moe_body(xb_ref, comb_ref, wgs_ref, wus_ref, wds_ref,
              wg_ref, wu_ref, wd_ref, out_ref):
    e = pl.program_id(0)
    t = pl.program_id(1)
    xb = xb_ref[...]

    @pl.when(e == 0)
    def _shared():
        hg = jnp.dot(xb, wgs_ref[...], preferred_element_type=jnp.float32)
        hu = jnp.dot(xb, wus_ref[...], preferred_element_type=jnp.float32)
        mid = (jax.nn.silu(hg) * hu).astype(jnp.bfloat16)
        out_ref[pl.ds(t * TBLK, TBLK), :] = jnp.dot(
            mid, wds_ref[...], preferred_element_type=jnp.float32)

    hg = jnp.dot(xb, wg_ref[0], preferred_element_type=jnp.float32)
    hu = jnp.dot(xb, wu_ref[0], preferred_element_type=jnp.float32)
    mid = jax.nn.silu(hg) * hu
    iota = jax.lax.broadcasted_iota(jnp.int32, (TBLK, E), 1)
    w_e = jnp.sum(jnp.where(iota == e, comb_ref[...], 0.0),
                  axis=1, keepdims=True)
    mid = (mid * w_e).astype(jnp.bfloat16)
    y = jnp.dot(mid, wd_ref[0], preferred_element_type=jnp.float32)
    out_ref[pl.ds(t * TBLK, TBLK), :] += y


def kernel(hidden_states, W_gate, Wg_s, Wu_s, Wd_s, Wg, Wu, Wd, expert_bias):
    b, s, h = hidden_states.shape
    x = hidden_states.reshape(T, H)

    comb = pl.pallas_call(
        _router_body,
        out_shape=jax.ShapeDtypeStruct((T, E), jnp.float32),
        in_specs=[
            pl.BlockSpec((T, H), lambda: (0, 0)),
            pl.BlockSpec((H, E), lambda: (0, 0)),
            pl.BlockSpec((1, E), lambda: (0, 0)),
        ],
        out_specs=pl.BlockSpec((T, E), lambda: (0, 0)),
    )(x, W_gate, expert_bias.reshape(1, E))

    xb = x.astype(jnp.bfloat16)
    out = pl.pallas_call(
        _moe_body,
        grid=(E, T // TBLK),
        out_shape=jax.ShapeDtypeStruct((T, H), jnp.float32),
        in_specs=[
            pl.BlockSpec((TBLK, H), lambda e, t: (t, 0)),
            pl.BlockSpec((TBLK, E), lambda e, t: (t, 0)),
            pl.BlockSpec((H, SI), lambda e, t: (0, 0)),
            pl.BlockSpec((H, SI), lambda e, t: (0, 0)),
            pl.BlockSpec((SI, H), lambda e, t: (0, 0)),
            pl.BlockSpec((1, H, INTER), lambda e, t: (e, 0, 0)),
            pl.BlockSpec((1, H, INTER), lambda e, t: (e, 0, 0)),
            pl.BlockSpec((1, INTER, H), lambda e, t: (e, 0, 0)),
        ],
        out_specs=pl.BlockSpec((T, H), lambda e, t: (0, 0)),
    )(xb, comb, Wg_s.astype(jnp.bfloat16), Wu_s.astype(jnp.bfloat16),
      Wd_s.astype(jnp.bfloat16), Wg.astype(jnp.bfloat16),
      Wu.astype(jnp.bfloat16), Wd.astype(jnp.bfloat16))

    return out.reshape(b, s, h)
